# baseline (speedup 1.0000x reference)
import functools

import jax
import jax.numpy as jnp
from jax import lax
from jax.experimental import pallas as pl
from jax.experimental.pallas import tpu as pltpu
from jax.experimental.pallas import tpu_sc as plsc

B = 16384
D = 32
L = 16
NW = 32
C = B // NW
TCH = 128
NCH = C // TCH
RPG = D // L
MARGIN = 1.0
EPS = 1e-6

_mesh = plsc.VectorSubcoreMesh(core_axis_name="c", subcore_axis_name="s")


def _sc_body(ent_hbm, rel_hbm, idx_hbm, out_hbm,
             iph, ipt, ipr, inh, int_, inr,
             gph, gpt, gpr, gnh, gnt, gnr,
             bph, bpt, bpr, bnh, bnt, bnr,
             sqp, sqn, sem):
    cid = lax.axis_index("c")
    sid = lax.axis_index("s")
    wid = sid * 2 + cid
    base = wid * C

    ivs = (iph, ipt, ipr, inh, int_, inr)
    gvs = (gph, gpt, gpr, gnh, gnt, gnr)
    bufs = (bph, bpt, bpr, bnh, bnt, bnr)
    tabs = (ent_hbm, ent_hbm, rel_hbm, ent_hbm, ent_hbm, rel_hbm)

    for seg in range(6):
        pltpu.sync_copy(idx_hbm.at[pl.ds(seg * B + base, C)], ivs[seg])

    def shift_body(j, carry):
        for seg in range(6):
            gvs[seg][pl.ds(j * L, L)] = ivs[seg][pl.ds(j * L, L)] >> 2
        return carry
    lax.fori_loop(0, C // L, shift_body, 0)

    for ch in range(NCH):
        cbase = ch * TCH
        cps = [
            pltpu.async_copy(tabs[seg].at[gvs[seg].at[pl.ds(cbase, TCH)]],
                             bufs[seg], sem)
            for seg in range(6)
        ]
        for cp in cps:
            cp.wait()

        def row_sq(buf_h, buf_r, buf_t, r, qh, qr, qt):
            h0 = buf_h[r, pl.ds(qh * D, L)]
            h1 = buf_h[r, pl.ds(qh * D + L, L)]
            r0 = buf_r[r, pl.ds(qr * D, L)]
            r1 = buf_r[r, pl.ds(qr * D + L, L)]
            t0 = buf_t[r, pl.ds(qt * D, L)]
            t1 = buf_t[r, pl.ds(qt * D + L, L)]
            d0 = h0 + r0 - t0 + EPS
            d1 = h1 + r1 - t1 + EPS
            return d0 * d0 + d1 * d1

        def group_body(g, carry):
            gr0 = cbase + g * L
            qph = iph[pl.ds(gr0, L)] & 3
            qpt = ipt[pl.ds(gr0, L)] & 3
            qpr = ipr[pl.ds(gr0, L)] & 3
            qnh = inh[pl.ds(gr0, L)] & 3
            qnt = int_[pl.ds(gr0, L)] & 3
            qnr = inr[pl.ds(gr0, L)] & 3
            for k in range(L):
                r = g * L + k
                sqp[pl.ds((gr0 + k) * L, L)] = row_sq(
                    bph, bpr, bpt, r, qph[k], qpr[k], qpt[k])
                sqn[pl.ds((gr0 + k) * L, L)] = row_sq(
                    bnh, bnr, bnt, r, qnh[k], qnr[k], qnt[k])
            return carry

        lax.fori_loop(0, TCH // L, group_body, 0)

    pltpu.sync_copy(sqp, out_hbm.at[pl.ds(base * L, C * L)])
    pltpu.sync_copy(sqn, out_hbm.at[pl.ds(B * L + base * L, C * L)])


_sc_distances = functools.partial(
    pl.kernel,
    out_type=jax.ShapeDtypeStruct((2 * B * L,), jnp.float32),
    mesh=_mesh,
    scratch_types=[
        pltpu.VMEM((C,), jnp.int32),
        pltpu.VMEM((C,), jnp.int32),
        pltpu.VMEM((C,), jnp.int32),
        pltpu.VMEM((C,), jnp.int32),
        pltpu.VMEM((C,), jnp.int32),
        pltpu.VMEM((C,), jnp.int32),
        pltpu.VMEM((C,), jnp.int32),
        pltpu.VMEM((C,), jnp.int32),
        pltpu.VMEM((C,), jnp.int32),
        pltpu.VMEM((C,), jnp.int32),
        pltpu.VMEM((C,), jnp.int32),
        pltpu.VMEM((C,), jnp.int32),
        pltpu.VMEM((TCH, 128), jnp.float32),
        pltpu.VMEM((TCH, 128), jnp.float32),
        pltpu.VMEM((TCH, 128), jnp.float32),
        pltpu.VMEM((TCH, 128), jnp.float32),
        pltpu.VMEM((TCH, 128), jnp.float32),
        pltpu.VMEM((TCH, 128), jnp.float32),
        pltpu.VMEM((C * L,), jnp.float32),
        pltpu.VMEM((C * L,), jnp.float32),
        pltpu.SemaphoreType.DMA,
    ],
)(_sc_body)

_ROWS = 2 * B * L // 128
_HALF = _ROWS // 2


def _finish_body(x_ref, o_ref):
    x = x_ref[...]
    i128 = lax.broadcasted_iota(jnp.int32, (128, 8), 0)
    i8 = lax.broadcasted_iota(jnp.int32, (128, 8), 1)
    s_mat = jnp.where(i128 // L == i8, 1.0, 0.0).astype(jnp.float32)
    d2p = jnp.dot(x[:_HALF], s_mat, preferred_element_type=jnp.float32)
    d2n = jnp.dot(x[_HALF:], s_mat, preferred_element_type=jnp.float32)
    m = jnp.sqrt(d2p) - jnp.sqrt(d2n) + MARGIN
    o_ref[...] = jnp.sum(jnp.maximum(m, 0.0), keepdims=True) * (1.0 / B)


_finish = pl.pallas_call(
    _finish_body,
    out_shape=jax.ShapeDtypeStruct((1, 1), jnp.float32),
)


def kernel(pos_x, neg_x, entity_weight, relation_weight):
    pos = pos_x.astype(jnp.int32)
    neg = neg_x.astype(jnp.int32)
    idx_flat = jnp.concatenate([
        pos[:, 0], pos[:, 2], pos[:, 1],
        neg[:, 0], neg[:, 2], neg[:, 1],
    ])
    ent4 = entity_weight.reshape(-1, 128)
    rel4 = relation_weight.reshape(-1, 128)
    sq = _sc_distances(ent4, rel4, idx_flat)
    return _finish(sq.reshape(_ROWS, 128))[0, 0]


# speedup vs baseline: 1.6800x; 1.6800x over previous
import functools

import jax
import jax.numpy as jnp
from jax import lax
from jax.experimental import pallas as pl
from jax.experimental.pallas import tpu as pltpu
from jax.experimental.pallas import tpu_sc as plsc

B = 16384
D = 32
L = 16
NW = 32
C = B // NW
TCH = 128
NCH = C // TCH
N = 1000000
NMAIN = 999936
GMAIN = NMAIN // 4
MARGIN = 1.0
EPS = 1e-6

_mesh = plsc.VectorSubcoreMesh(core_axis_name="c", subcore_axis_name="s")


def _sc_body(ent_hbm, rel_hbm, etail_hbm, rtail_hbm, idx_hbm, out_hbm,
             iph, ipt, ipr, inh, int_, inr,
             gph, gpt, gpr, gnh, gnt, gnr,
             bph, bpt, bpr, bnh, bnt, bnr,
             etv, rtv, sqp, sqn, sem):
    cid = lax.axis_index("c")
    sid = lax.axis_index("s")
    wid = sid * 2 + cid
    base = wid * C

    ivs = (iph, ipt, ipr, inh, int_, inr)
    gvs = (gph, gpt, gpr, gnh, gnt, gnr)
    bufs = (bph, bpt, bpr, bnh, bnt, bnr)
    tabs = (ent_hbm, ent_hbm, rel_hbm, ent_hbm, ent_hbm, rel_hbm)

    pltpu.sync_copy(etail_hbm, etv)
    pltpu.sync_copy(rtail_hbm, rtv)

    for seg in range(6):
        pltpu.sync_copy(idx_hbm.at[pl.ds(seg * B + base, C)], ivs[seg])

    def shift_body(j, carry):
        for seg in range(6):
            iv = ivs[seg][pl.ds(j * L, L)]
            g = ((iv >> 9) << 7) | (iv & 127)
            gvs[seg][pl.ds(j * L, L)] = jnp.minimum(g, GMAIN - 1)
        return carry
    lax.fori_loop(0, C // L, shift_body, 0)

    def pick(buf, tv, iv_chunk, k, r):
        idx_s = iv_chunk[k]
        q32m = ((idx_s >> 7) & 3) * D
        e_t = idx_s - NMAIN
        tr = jnp.clip(e_t >> 2, 0, 15)
        q32t = (e_t & 3) * D
        w = jnp.where(idx_s >= NMAIN, 1.0, 0.0)
        m0 = buf[r, pl.ds(q32m, L)]
        m1 = buf[r, pl.ds(q32m + L, L)]
        t0 = tv[tr, pl.ds(q32t, L)]
        t1 = tv[tr, pl.ds(q32t + L, L)]
        return m0 + w * (t0 - m0), m1 + w * (t1 - m1)

    def chunk_body(ch, carry):
        cbase = ch * TCH
        cps = [
            pltpu.async_copy(tabs[seg].at[gvs[seg].at[pl.ds(cbase, TCH)]],
                             bufs[seg], sem)
            for seg in range(6)
        ]
        for cp in cps:
            cp.wait()

        def group_body(g, carry2):
            gr0 = cbase + g * L
            ivc = [ivs[seg][pl.ds(gr0, L)] for seg in range(6)]
            for k in range(L):
                r = g * L + k
                h0, h1 = pick(bph, etv, ivc[0], k, r)
                t0, t1 = pick(bpt, etv, ivc[1], k, r)
                r0, r1 = pick(bpr, rtv, ivc[2], k, r)
                d0 = h0 + r0 - t0 + EPS
                d1 = h1 + r1 - t1 + EPS
                sqp[pl.ds((gr0 + k) * L, L)] = d0 * d0 + d1 * d1
                h0, h1 = pick(bnh, etv, ivc[3], k, r)
                t0, t1 = pick(bnt, etv, ivc[4], k, r)
                r0, r1 = pick(bnr, rtv, ivc[5], k, r)
                d0 = h0 + r0 - t0 + EPS
                d1 = h1 + r1 - t1 + EPS
                sqn[pl.ds((gr0 + k) * L, L)] = d0 * d0 + d1 * d1
            return carry2

        lax.fori_loop(0, TCH // L, group_body, 0)
        return carry

    lax.fori_loop(0, NCH, chunk_body, 0)

    pltpu.sync_copy(sqp, out_hbm.at[pl.ds(base * L, C * L)])
    pltpu.sync_copy(sqn, out_hbm.at[pl.ds(B * L + base * L, C * L)])


_sc_distances = functools.partial(
    pl.kernel,
    out_type=jax.ShapeDtypeStruct((2 * B * L,), jnp.float32),
    mesh=_mesh,
    scratch_types=(
        [pltpu.VMEM((C,), jnp.int32) for _ in range(6)]
        + [pltpu.VMEM((C,), jnp.int32) for _ in range(6)]
        + [pltpu.VMEM((TCH, 128), jnp.float32) for _ in range(6)]
        + [pltpu.VMEM((16, 128), jnp.float32),
           pltpu.VMEM((16, 128), jnp.float32),
           pltpu.VMEM((C * L,), jnp.float32),
           pltpu.VMEM((C * L,), jnp.float32),
           pltpu.SemaphoreType.DMA]
    ),
)(_sc_body)

_W = 15872
_NBLK = NMAIN // _W


def _t_body(x_ref, o_ref):
    for j in range(_W // 512):
        for q in range(4):
            o_ref[j * 128:(j + 1) * 128, q * D:(q + 1) * D] = jnp.transpose(
                x_ref[:, j * 512 + q * 128: j * 512 + (q + 1) * 128])


_to_rows = pl.pallas_call(
    _t_body,
    grid=(_NBLK,),
    in_specs=[pl.BlockSpec((32, _W), lambda j: (0, j))],
    out_specs=pl.BlockSpec((_W // 4, 128), lambda j: (j, 0)),
    out_shape=jax.ShapeDtypeStruct((GMAIN, 128), jnp.float32),
)

_ROWS = 2 * B * L // 128
_HALF = _ROWS // 2


def _finish_body(x_ref, o_ref):
    x = x_ref[...]
    i128 = lax.broadcasted_iota(jnp.int32, (128, 8), 0)
    i8 = lax.broadcasted_iota(jnp.int32, (128, 8), 1)
    s_mat = jnp.where(i128 // L == i8, 1.0, 0.0).astype(jnp.float32)
    d2p = jnp.dot(x[:_HALF], s_mat, preferred_element_type=jnp.float32)
    d2n = jnp.dot(x[_HALF:], s_mat, preferred_element_type=jnp.float32)
    m = jnp.sqrt(d2p) - jnp.sqrt(d2n) + MARGIN
    o_ref[...] = jnp.sum(jnp.maximum(m, 0.0), keepdims=True) * (1.0 / B)


_finish = pl.pallas_call(
    _finish_body,
    out_shape=jax.ShapeDtypeStruct((1, 1), jnp.float32),
)


def kernel(pos_x, neg_x, entity_weight, relation_weight):
    pos = pos_x.astype(jnp.int32)
    neg = neg_x.astype(jnp.int32)
    idx_flat = jnp.concatenate([
        pos[:, 0], pos[:, 2], pos[:, 1],
        neg[:, 0], neg[:, 2], neg[:, 1],
    ])
    ent4 = _to_rows(entity_weight.T)
    rel4 = _to_rows(relation_weight.T)
    etail = entity_weight[NMAIN:].reshape(16, 128)
    rtail = relation_weight[NMAIN:].reshape(16, 128)
    sq = _sc_distances(ent4, rel4, etail, rtail, idx_flat)
    return _finish(sq.reshape(_ROWS, 128))[0, 0]


# speedup vs baseline: 1.6810x; 1.0006x over previous
import functools

import jax
import jax.numpy as jnp
from jax import lax
from jax.experimental import pallas as pl
from jax.experimental.pallas import tpu as pltpu
from jax.experimental.pallas import tpu_sc as plsc

B = 16384
D = 32
L = 16
NW = 32
C = B // NW
TCH = 128
NCH = C // TCH
N = 1000000
NMAIN = 999936
GMAIN = NMAIN // 4
MARGIN = 1.0
EPS = 1e-6

_mesh = plsc.VectorSubcoreMesh(core_axis_name="c", subcore_axis_name="s")


def _sc_body(ent_hbm, rel_hbm, etail_hbm, rtail_hbm, idx_hbm, out_hbm,
             iph, ipt, ipr, inh, int_, inr,
             gph, gpt, gpr, gnh, gnt, gnr,
             bph, bpt, bpr, bnh, bnt, bnr,
             etv, rtv, sqp, sqn, sem):
    cid = lax.axis_index("c")
    sid = lax.axis_index("s")
    wid = sid * 2 + cid
    base = wid * C

    ivs = (iph, ipt, ipr, inh, int_, inr)
    gvs = (gph, gpt, gpr, gnh, gnt, gnr)
    bufs = (bph, bpt, bpr, bnh, bnt, bnr)
    tabs = (ent_hbm, ent_hbm, rel_hbm, ent_hbm, ent_hbm, rel_hbm)

    pltpu.sync_copy(etail_hbm, etv)
    pltpu.sync_copy(rtail_hbm, rtv)

    for seg in range(6):
        pltpu.sync_copy(idx_hbm.at[pl.ds(seg * B + base, C)], ivs[seg])

    def shift_body(j, carry):
        for seg in range(6):
            iv = ivs[seg][pl.ds(j * L, L)]
            g = ((iv >> 9) << 7) | (iv & 127)
            gvs[seg][pl.ds(j * L, L)] = jnp.minimum(g, GMAIN - 1)
        return carry
    lax.fori_loop(0, C // L, shift_body, 0)

    def pick(buf, tv, iv_chunk, k, r):
        idx_s = iv_chunk[k]
        q32m = ((idx_s >> 7) & 3) * D
        e_t = idx_s - NMAIN
        tr = jnp.clip(e_t >> 2, 0, 15)
        q32t = (e_t & 3) * D
        w = jnp.where(idx_s >= NMAIN, 1.0, 0.0)
        m0 = buf[r, pl.ds(q32m, L)]
        m1 = buf[r, pl.ds(q32m + L, L)]
        t0 = tv[tr, pl.ds(q32t, L)]
        t1 = tv[tr, pl.ds(q32t + L, L)]
        return m0 + w * (t0 - m0), m1 + w * (t1 - m1)

    def chunk_body(ch, carry):
        cbase = ch * TCH
        cps = [
            pltpu.async_copy(tabs[seg].at[gvs[seg].at[pl.ds(cbase, TCH)]],
                             bufs[seg], sem)
            for seg in range(6)
        ]
        for cp in cps:
            cp.wait()

        def group_body(g, carry2):
            gr0 = cbase + g * L
            ivc = [ivs[seg][pl.ds(gr0, L)] for seg in range(6)]
            for k in range(L):
                r = g * L + k
                h0, h1 = pick(bph, etv, ivc[0], k, r)
                t0, t1 = pick(bpt, etv, ivc[1], k, r)
                r0, r1 = pick(bpr, rtv, ivc[2], k, r)
                d0 = h0 + r0 - t0 + EPS
                d1 = h1 + r1 - t1 + EPS
                sqp[pl.ds((gr0 + k) * L, L)] = d0 * d0 + d1 * d1
                h0, h1 = pick(bnh, etv, ivc[3], k, r)
                t0, t1 = pick(bnt, etv, ivc[4], k, r)
                r0, r1 = pick(bnr, rtv, ivc[5], k, r)
                d0 = h0 + r0 - t0 + EPS
                d1 = h1 + r1 - t1 + EPS
                sqn[pl.ds((gr0 + k) * L, L)] = d0 * d0 + d1 * d1
            return carry2

        lax.fori_loop(0, TCH // L, group_body, 0)
        return carry

    lax.fori_loop(0, NCH, chunk_body, 0)

    pltpu.sync_copy(sqp, out_hbm.at[pl.ds(base * L, C * L)])
    pltpu.sync_copy(sqn, out_hbm.at[pl.ds(B * L + base * L, C * L)])


_sc_distances = functools.partial(
    pl.kernel,
    out_type=jax.ShapeDtypeStruct((2 * B * L,), jnp.float32),
    mesh=_mesh,
    scratch_types=(
        [pltpu.VMEM((C,), jnp.int32) for _ in range(6)]
        + [pltpu.VMEM((C,), jnp.int32) for _ in range(6)]
        + [pltpu.VMEM((TCH, 128), jnp.float32) for _ in range(6)]
        + [pltpu.VMEM((16, 128), jnp.float32),
           pltpu.VMEM((16, 128), jnp.float32),
           pltpu.VMEM((C * L,), jnp.float32),
           pltpu.VMEM((C * L,), jnp.float32),
           pltpu.SemaphoreType.DMA]
    ),
)(_sc_body)

_W = 15872
_NBLK = NMAIN // _W


def _t_body(x_ref, o_ref):
    y = jnp.transpose(x_ref[...])
    y4 = y.reshape(_W // 512, 4, 128, D)
    for q in range(4):
        o_ref[:, q * D:(q + 1) * D] = y4[:, q].reshape(_W // 4, D)


_to_rows = pl.pallas_call(
    _t_body,
    grid=(_NBLK,),
    in_specs=[pl.BlockSpec((32, _W), lambda j: (0, j))],
    out_specs=pl.BlockSpec((_W // 4, 128), lambda j: (j, 0)),
    out_shape=jax.ShapeDtypeStruct((GMAIN, 128), jnp.float32),
)

_ROWS = 2 * B * L // 128
_HALF = _ROWS // 2


def _finish_body(x_ref, o_ref):
    x = x_ref[...]
    i128 = lax.broadcasted_iota(jnp.int32, (128, 8), 0)
    i8 = lax.broadcasted_iota(jnp.int32, (128, 8), 1)
    s_mat = jnp.where(i128 // L == i8, 1.0, 0.0).astype(jnp.float32)
    d2p = jnp.dot(x[:_HALF], s_mat, preferred_element_type=jnp.float32)
    d2n = jnp.dot(x[_HALF:], s_mat, preferred_element_type=jnp.float32)
    m = jnp.sqrt(d2p) - jnp.sqrt(d2n) + MARGIN
    o_ref[...] = jnp.sum(jnp.maximum(m, 0.0), keepdims=True) * (1.0 / B)


_finish = pl.pallas_call(
    _finish_body,
    out_shape=jax.ShapeDtypeStruct((1, 1), jnp.float32),
)


def kernel(pos_x, neg_x, entity_weight, relation_weight):
    pos = pos_x.astype(jnp.int32)
    neg = neg_x.astype(jnp.int32)
    idx_flat = jnp.concatenate([
        pos[:, 0], pos[:, 2], pos[:, 1],
        neg[:, 0], neg[:, 2], neg[:, 1],
    ])
    ent4 = _to_rows(entity_weight.T)
    rel4 = _to_rows(relation_weight.T)
    etail = entity_weight[NMAIN:].reshape(16, 128)
    rtail = relation_weight[NMAIN:].reshape(16, 128)
    sq = _sc_distances(ent4, rel4, etail, rtail, idx_flat)
    return _finish(sq.reshape(_ROWS, 128))[0, 0]
